# baseline (speedup 1.0000x reference)
import functools

import jax
import jax.numpy as jnp
from jax.experimental import pallas as pl
from jax.experimental.pallas import tpu as pltpu

_B = 16


def _fused_kernel(heads, slices, dh, nb, xb_ref, bcol_ref, wfx_ref, bfx_ref,
                  wx_ref, bx_ref, wsl_ref, bsl_ref, invg_ref, gsum_ref,
                  wq_ref, wk_ref, wv_ref, wout_ref, bout_ref,
                  out_ref, sw_ref, acc_ref, att2_ref):
    p = pl.program_id(0)
    i = pl.program_id(1)
    bs = _B * slices
    scale = float(dh) ** -0.5

    def onehot512(bcol):
        bid = jax.lax.broadcasted_iota(jnp.int32, (1, bs), 1) // slices
        return (bcol == bid.astype(jnp.bfloat16)).astype(jnp.bfloat16)

    @pl.when(p == 0)
    def _pool():
        @pl.when(i == 0)
        def _init():
            acc_ref[...] = jnp.zeros_like(acc_ref)

        xb = xb_ref[...]
        fx = jnp.dot(xb, wfx_ref[...], preferred_element_type=jnp.float32) + bfx_ref[...]
        xm = (jnp.dot(xb, wx_ref[...], preferred_element_type=jnp.float32)
              + bx_ref[...]).astype(jnp.bfloat16)
        logits = (jnp.dot(xm, wsl_ref[...], preferred_element_type=jnp.float32)
                  + bsl_ref[...]) * invg_ref[...]
        m = jnp.max(logits, axis=-1, keepdims=True)
        e = jnp.exp(logits - m).astype(jnp.bfloat16)
        denom = jnp.dot(e, gsum_ref[...], preferred_element_type=jnp.float32)
        sw = (e / denom).astype(jnp.bfloat16)
        sw_ref[pl.ds(i * nb, nb), :] = sw
        fxb = fx.astype(jnp.bfloat16)
        onehot = onehot512(bcol_ref[...])
        ones = jnp.ones((nb, 1), jnp.bfloat16)
        for h in range(heads):
            sw_h = sw[:, h * slices:(h + 1) * slices]
            comb = onehot * jnp.concatenate([sw_h] * _B, axis=1)
            fxa = jnp.concatenate([fxb[:, h * slices:(h + 1) * slices], ones],
                                  axis=1)
            res = jax.lax.dot_general(fxa, comb, (((0,), (0,)), ((), ())),
                                      preferred_element_type=jnp.float32)
            acc_ref[h] = acc_ref[h] + res

    @pl.when((p == 1) & (i == 0))
    def _attend():
        r = jax.lax.broadcasted_iota(jnp.int32, (bs, bs), 0) // slices
        c = jax.lax.broadcasted_iota(jnp.int32, (bs, bs), 1) // slices
        same_graph = r == c
        for h in range(heads):
            a = acc_ref[h]
            tok_t = a[:dh, :] / (a[dh:dh + 1, :] + 1e-5)
            q_t = jnp.dot(wq_ref[...], tok_t, preferred_element_type=jnp.float32)
            k_t = jnp.dot(wk_ref[...], tok_t, preferred_element_type=jnp.float32)
            v_t = jnp.dot(wv_ref[...], tok_t, preferred_element_type=jnp.float32)
            dots = jax.lax.dot_general(q_t, k_t, (((0,), (0,)), ((), ())),
                                       preferred_element_type=jnp.float32)
            dots = jnp.where(same_graph, dots * scale, -1e30)
            mx = jnp.max(dots, axis=-1, keepdims=True)
            e = jnp.exp(dots - mx)
            attn = e / jnp.sum(e, axis=-1, keepdims=True)
            att_t = jax.lax.dot_general(v_t, attn, (((1,), (1,)), ((), ())),
                                        preferred_element_type=jnp.float32)
            att2_ref[h] = jax.lax.dot_general(
                att_t, wout_ref[h * dh:(h + 1) * dh, :],
                (((0,), (0,)), ((), ())),
                preferred_element_type=jnp.float32).astype(jnp.bfloat16)

    @pl.when(p == 1)
    def _readback():
        sw = sw_ref[pl.ds(i * nb, nb), :]
        onehot = onehot512(bcol_ref[...])
        o = jnp.zeros(out_ref.shape, jnp.float32) + bout_ref[...]
        for h in range(heads):
            sw_h = sw[:, h * slices:(h + 1) * slices]
            comb = onehot * jnp.concatenate([sw_h] * _B, axis=1)
            o = o + jnp.dot(comb, att2_ref[h], preferred_element_type=jnp.float32)
        out_ref[...] = o


def kernel(x, batch, W_fx, b_fx, W_x, b_x, W_slice, b_slice, Wq, Wk, Wv,
           W_out, b_out, g_temp):
    n, dim = x.shape
    heads = g_temp.shape[1]
    inner = W_fx.shape[0]
    dh = inner // heads
    slices = W_slice.shape[0]
    bs = _B * slices

    nb = 2000
    grid = (2, n // nb)

    xbf = x.astype(jnp.bfloat16)
    bcol = batch.astype(jnp.bfloat16).reshape(n, 1)
    eye_h = jnp.eye(heads, dtype=jnp.float32)
    wsl_t = jnp.einsum('hk,sd->hdks', eye_h, W_slice).reshape(inner, heads * slices)
    bsl = jnp.tile(b_slice, heads).reshape(1, heads * slices)
    invg = jnp.repeat(1.0 / g_temp.reshape(heads), slices).reshape(1, heads * slices)
    gs_i = jnp.arange(heads * slices) // slices
    gsum = (gs_i[:, None] == gs_i[None, :]).astype(jnp.float32)

    blk = lambda r, c: pl.BlockSpec((r, c), lambda p, i: (i, 0))
    full = lambda r, c: pl.BlockSpec((r, c), lambda p, i: (0, 0))

    out = pl.pallas_call(
        functools.partial(_fused_kernel, heads, slices, dh, nb),
        grid=grid,
        in_specs=[
            blk(nb, dim),
            blk(nb, 1),
            full(dim, inner), full(1, inner), full(dim, inner), full(1, inner),
            full(inner, heads * slices), full(1, heads * slices),
            full(1, heads * slices), full(heads * slices, heads * slices),
            full(dh, dh), full(dh, dh), full(dh, dh),
            full(inner, dim), full(1, dim),
        ],
        out_specs=blk(nb, dim),
        out_shape=jax.ShapeDtypeStruct((n, dim), jnp.float32),
        scratch_shapes=[
            pltpu.VMEM((n, heads * slices), jnp.bfloat16),
            pltpu.VMEM((heads, dh + 1, bs), jnp.float32),
            pltpu.VMEM((heads, bs, dim), jnp.bfloat16),
        ],
    )(xbf, bcol, W_fx.T.astype(jnp.bfloat16), b_fx.reshape(1, inner),
      W_x.T.astype(jnp.bfloat16), b_x.reshape(1, inner),
      wsl_t.astype(jnp.bfloat16), bsl, invg, gsum.astype(jnp.bfloat16),
      Wq, Wk, Wv, W_out.T, b_out.reshape(1, dim))
    return out
